# Initial kernel scaffold; baseline (speedup 1.0000x reference)
import jax
import jax.numpy as jnp
from jax.experimental import pallas as pl


def kernel(node_feats, edge_feats, edge_index, dist, Wm, bm, Wa, ba):
    raise NotImplementedError("write your pallas kernel here")


# speedup vs baseline: 1.7157x; 1.7157x over previous
import functools

import jax
import jax.numpy as jnp
from jax import lax
from jax.experimental import pallas as pl
from jax.experimental.pallas import tpu as pltpu
from jax.experimental.pallas import tpu_sc as plsc

CUTOFF = 10.0
D = 128
NC, NS, L = 2, 16, 16
NW = NC * NS
CHUNK = 96
NN = 10000
AGG_PAD = 10240
ROW_BLK = 400
EF_BLK = 6720


def _envelope_body(d_ref, c_ref):
    d = d_ref[...]
    c = 0.5 * (jnp.cos(d * (jnp.pi / CUTOFF)) + 1.0)
    c_ref[...] = c * (d < CUTOFF).astype(jnp.float32)


def _epre_body(ef_ref, w_ref, b_ref, o_ref):
    o_ref[...] = (
        jnp.dot(ef_ref[...], w_ref[...], preferred_element_type=jnp.float32)
        + b_ref[...]
    )


def _xsxd_body(x_ref, ws_ref, wd_ref, xs_ref, xd_ref):
    x = x_ref[...]
    xs_ref[...] = jnp.dot(x, ws_ref[...], preferred_element_type=jnp.float32)
    xd_ref[...] = jnp.dot(x, wd_ref[...], preferred_element_type=jnp.float32)


def _update_body(x_ref, a0_ref, a1_ref, wx_ref, wg_ref, b_ref, o_ref):
    h = (
        jnp.dot(x_ref[...], wx_ref[...], preferred_element_type=jnp.float32)
        + jnp.dot(a0_ref[...] + a1_ref[...], wg_ref[...],
                  preferred_element_type=jnp.float32)
        + b_ref[...]
    )
    o_ref[...] = jnp.maximum(h, 0.0)


def _edge_body(xs_hbm, xd_hbm, epre_hbm, src_hbm, dst_hbm, c_hbm, zero_hbm,
               out_hbm, sbuf, dbuf, ebuf, sidx, didx, cbuf, agg_sh, sem,
               *, n_chunks):
    cid = lax.axis_index("c")
    sid = lax.axis_index("s")
    wid = cid * NS + sid
    nps = AGG_PAD // NS

    row0 = sid * nps
    pltpu.sync_copy(zero_hbm.at[pl.ds(row0, nps)], agg_sh.at[pl.ds(row0, nps)])
    plsc.subcore_barrier()

    base = wid * (n_chunks * CHUNK)

    def chunk_body(c, _):
        off = base + c * CHUNK
        pltpu.sync_copy(src_hbm.at[pl.ds(off, CHUNK)], sidx)
        pltpu.sync_copy(dst_hbm.at[pl.ds(off, CHUNK)], didx)
        pltpu.sync_copy(c_hbm.at[pl.ds(off, CHUNK)], cbuf)
        pltpu.sync_copy(epre_hbm.at[pl.ds(off, CHUNK)], ebuf)
        pltpu.async_copy(xs_hbm.at[sidx], sbuf, sem).wait()
        pltpu.async_copy(xd_hbm.at[didx], dbuf, sem).wait()

        def group_body(g, _):
            cv16 = cbuf[pl.ds(g * L, L)]

            def edge_body(j, _):
                e = g * L + j
                cv = lax.gather(
                    cv16, jnp.full((L, 1), j, jnp.int32),
                    lax.GatherDimensionNumbers(offset_dims=(),
                                               collapsed_slice_dims=(0,),
                                               start_index_map=(0,)),
                    slice_sizes=(1,),
                    mode=lax.GatherScatterMode.PROMISE_IN_BOUNDS)
                for f in range(D // L):
                    sl = pl.ds(f * L, L)
                    v = sbuf[e, sl] + dbuf[e, sl] + ebuf[e, sl]
                    ebuf[e, sl] = jnp.maximum(v, 0.0) * cv
                return 0

            lax.fori_loop(0, L, edge_body, 0)
            return 0

        lax.fori_loop(0, CHUNK // L, group_body, 0)
        pltpu.sync_copy(ebuf, agg_sh.at[didx], add=True)
        return 0

    lax.fori_loop(0, n_chunks, chunk_body, 0)
    plsc.subcore_barrier()
    pltpu.sync_copy(agg_sh.at[pl.ds(row0, nps)],
                    out_hbm.at[cid, pl.ds(row0, nps)])


def _make_edge_kernel(n_chunks):
    mesh = plsc.VectorSubcoreMesh(core_axis_name="c", subcore_axis_name="s",
                                  num_cores=NC, num_subcores=NS)
    return pl.kernel(
        functools.partial(_edge_body, n_chunks=n_chunks),
        out_type=jax.ShapeDtypeStruct((NC, AGG_PAD, D), jnp.float32),
        mesh=mesh,
        scratch_types=[
            pltpu.VMEM((CHUNK, D), jnp.float32),
            pltpu.VMEM((CHUNK, D), jnp.float32),
            pltpu.VMEM((CHUNK, D), jnp.float32),
            pltpu.VMEM((CHUNK,), jnp.int32),
            pltpu.VMEM((CHUNK,), jnp.int32),
            pltpu.VMEM((CHUNK,), jnp.float32),
            pltpu.VMEM_SHARED((AGG_PAD, D), jnp.float32),
            pltpu.SemaphoreType.DMA,
        ],
    )


def kernel(node_feats, edge_feats, edge_index, dist, Wm, bm, Wa, ba):
    n, d_in = node_feats.shape
    e, d_edge = edge_feats.shape
    f32 = jnp.float32

    e_pad = -(-e // (NW * CHUNK)) * (NW * CHUNK)
    n_chunks = e_pad // (NW * CHUNK)

    x = node_feats
    ef = jnp.pad(edge_feats, ((0, e_pad - e), (0, 0)))
    src = jnp.pad(edge_index[0], (0, e_pad - e))
    dst = jnp.pad(edge_index[1], (0, e_pad - e))
    distp = jnp.pad(dist, (0, e_pad - e), constant_values=2.0 * CUTOFF)
    zeros = jnp.zeros((AGG_PAD, D), f32)

    env = pl.pallas_call(
        _envelope_body,
        out_shape=jax.ShapeDtypeStruct((e_pad // D, D), f32),
    )(distp.reshape(e_pad // D, D))
    env = env.reshape(e_pad)

    n_row_blocks = NN // ROW_BLK
    xsxd_call = pl.pallas_call(
        _xsxd_body,
        grid=(n_row_blocks,),
        in_specs=[
            pl.BlockSpec((ROW_BLK, D), lambda i: (i, 0)),
            pl.BlockSpec((D, D), lambda i: (0, 0)),
            pl.BlockSpec((D, D), lambda i: (0, 0)),
        ],
        out_specs=[
            pl.BlockSpec((ROW_BLK, D), lambda i: (i, 0)),
            pl.BlockSpec((ROW_BLK, D), lambda i: (i, 0)),
        ],
        out_shape=[jax.ShapeDtypeStruct((NN, D), f32)] * 2,
    )

    epre_call = pl.pallas_call(
        _epre_body,
        grid=(e_pad // EF_BLK,),
        in_specs=[
            pl.BlockSpec((EF_BLK, d_edge), lambda i: (i, 0)),
            pl.BlockSpec((d_edge, D), lambda i: (0, 0)),
            pl.BlockSpec((1, D), lambda i: (0, 0)),
        ],
        out_specs=pl.BlockSpec((EF_BLK, D), lambda i: (i, 0)),
        out_shape=jax.ShapeDtypeStruct((e_pad, D), f32),
    )

    update_call = pl.pallas_call(
        _update_body,
        grid=(n_row_blocks,),
        in_specs=[
            pl.BlockSpec((ROW_BLK, D), lambda i: (i, 0)),
            pl.BlockSpec((ROW_BLK, D), lambda i: (i, 0)),
            pl.BlockSpec((ROW_BLK, D), lambda i: (i, 0)),
            pl.BlockSpec((D, D), lambda i: (0, 0)),
            pl.BlockSpec((D, D), lambda i: (0, 0)),
            pl.BlockSpec((1, D), lambda i: (0, 0)),
        ],
        out_specs=pl.BlockSpec((ROW_BLK, D), lambda i: (i, 0)),
        out_shape=jax.ShapeDtypeStruct((NN, D), f32),
    )

    edge_call = _make_edge_kernel(n_chunks)

    num_layers = Wm.shape[0]
    for l in range(num_layers):
        ws, wd, we = Wm[l, :D], Wm[l, D:2 * D], Wm[l, 2 * D:]
        epre = epre_call(ef, we, bm[l][None])
        xs, xd = xsxd_call(x, ws, wd)
        agg2 = edge_call(xs, xd, epre, src, dst, env, zeros)
        x = update_call(x, agg2[0], agg2[1], Wa[l, :D], Wa[l, D:],
                        ba[l][None])
    return x
